# baseline (speedup 1.0000x reference)
import jax
import jax.numpy as jnp
import numpy as np
from jax import lax
from jax.experimental import pallas as pl
from jax.experimental.pallas import tpu as pltpu
from jax.experimental.pallas import tpu_sc as plsc

_M = 32
_N = 2
_SUBSET = 8
_B = 128
_D = 528
_L = 45

_NC, _NS = 2, 16
_NW = _NC * _NS
_NRC = 4
_NBC = 8
_RPC = _D // _NRC
_RPAD = 144
_NBIN = _L + 1
_ACC = _L * 16


def _fock_states(m, n):
    if n == 0:
        yield (0,) * m
        return
    if m == 1:
        yield (n,)
        return
    for i in range(n + 1):
        for s in _fock_states(m - 1, n - i):
            yield (i,) + s


def _build_tables():
    all_states = list(_fock_states(_M, _N))
    reduced = []
    for i in range(max(0, _SUBSET - _M + _N), _N + 1):
        reduced += list(_fock_states(_SUBSET, i))
    binmap = np.array([reduced.index(s[:_SUBSET]) for s in all_states],
                      dtype=np.int64)
    gidx = np.zeros((_NRC * _RPAD,), np.int32)
    bins = np.full((_NRC * _RPAD,), _L, np.int32)
    for c in range(_NRC):
        for t in range(_RPC):
            i = c * _RPC + t
            gidx[c * _RPAD + t] = (_D + 1) * i
            bins[c * _RPAD + t] = binmap[i]
    return gidx, bins


_GIDX_NP, _BINS_NP = _build_tables()


def _sc_body(diag_hbm, gidx_hbm, bins_hbm, out_hbm,
             idx_v, bins_v, vals_v, acc_v, sem):
    wid = lax.axis_index("s") * _NC + lax.axis_index("c")
    rc = wid // _NBC
    bc = wid - rc * _NBC
    boff = pl.multiple_of(bc * 16, 16)
    toff = pl.multiple_of(rc * _RPAD, 8)

    pltpu.sync_copy(gidx_hbm.at[pl.ds(toff, _RPAD)], idx_v)
    pltpu.sync_copy(bins_hbm.at[pl.ds(toff, _RPAD)],
                    bins_v.at[pl.ds(0, _RPAD)])
    cp0 = pltpu.async_copy(diag_hbm.at[idx_v.at[pl.ds(0, 72)]],
                           vals_v.at[pl.ds(0, 72)], sem)
    cp1 = pltpu.async_copy(diag_hbm.at[idx_v.at[pl.ds(72, 72)]],
                           vals_v.at[pl.ds(72, 72)], sem)

    zeros = jnp.zeros((16,), jnp.float32)

    def _zero_acc(k, carry):
        acc_v[pl.ds(pl.multiple_of(k * 16, 16), 16)] = zeros
        return carry
    lax.fori_loop(0, _NBIN, _zero_acc, 0)

    cp0.wait()
    cp1.wait()

    def _accum(t, carry):
        b = bins_v[pl.ds(t, 16)][0]
        val = vals_v[t, pl.ds(boff, 16)]
        off = pl.multiple_of(b * 16, 16)
        acc_v[pl.ds(off, 16)] = acc_v[pl.ds(off, 16)] + jnp.abs(val)
        return carry
    lax.fori_loop(0, _RPAD, _accum, 0, unroll=4)

    pltpu.sync_copy(acc_v.at[pl.ds(0, _ACC)],
                    out_hbm.at[pl.ds(wid * _ACC, _ACC)])


@jax.jit
def _partial_measure(diag_tab, gidx, bins):
    mesh = plsc.VectorSubcoreMesh(core_axis_name="c", subcore_axis_name="s",
                                  num_cores=_NC, num_subcores=_NS)
    parts = pl.kernel(
        _sc_body,
        out_type=jax.ShapeDtypeStruct((_NW * _ACC,), jnp.float32),
        mesh=mesh,
        compiler_params=pltpu.CompilerParams(needs_layout_passes=False,
                                             use_tc_tiling_on_sc=True),
        scratch_types=[
            pltpu.VMEM((_RPAD,), jnp.int32),
            pltpu.VMEM((_RPAD + 16,), jnp.int32),
            pltpu.VMEM((_RPAD, _B), jnp.float32),
            pltpu.VMEM((_NBIN * 16,), jnp.float32),
            pltpu.SemaphoreType.DMA,
        ],
    )(diag_tab, gidx, bins)
    p = parts.reshape(_NRC, _NBC, _L, 16).sum(0)
    return p.transpose(0, 2, 1).reshape(_B, _L)


def kernel(rho):
    diag_tab = jnp.transpose(rho, (1, 2, 0)).reshape(_D * _D, _B)
    return _partial_measure(diag_tab, jnp.asarray(_GIDX_NP),
                            jnp.asarray(_BINS_NP))


# speedup vs baseline: 1.0359x; 1.0359x over previous
import jax
import jax.numpy as jnp
import numpy as np
from jax import lax
from jax.experimental import pallas as pl
from jax.experimental.pallas import tpu as pltpu
from jax.experimental.pallas import tpu_sc as plsc

_M = 32
_N = 2
_SUBSET = 8
_B = 128
_D = 528
_L = 45

_NC, _NS = 2, 16
_NW = _NC * _NS
_NRC = 4
_NBC = 8
_RPC = _D // _NRC
_RPAD = 144
_SB = 24
_ACC = _L * 16


def _fock_states(m, n):
    if n == 0:
        yield (0,) * m
        return
    if m == 1:
        yield (n,)
        return
    for i in range(n + 1):
        for s in _fock_states(m - 1, n - i):
            yield (i,) + s


def _build_tables():
    all_states = list(_fock_states(_M, _N))
    reduced = []
    for i in range(max(0, _SUBSET - _M + _N), _N + 1):
        reduced += list(_fock_states(_SUBSET, i))
    binmap = np.array([reduced.index(s[:_SUBSET]) for s in all_states],
                      dtype=np.int64)
    rows_of = {b: list(np.where(binmap == b)[0]) for b in range(_L)}
    big = [b for b in range(_L) if len(rows_of[b]) == 300]
    mid = sorted(b for b in range(_L) if len(rows_of[b]) == 24)
    single = sorted(b for b in range(_L) if len(rows_of[b]) == 1)
    assert len(big) == 1 and len(mid) == 8 and len(single) == 36

    gidx = np.zeros((_NRC, _RPAD), np.int32)
    sbins = np.full((_NRC, _SB), _L, np.int32)
    for c in range(_NRC):
        pos = 0
        for r in rows_of[big[0]][c * 75:(c + 1) * 75]:
            gidx[c, pos] = (_D + 1) * r
            pos += 1
        for b in mid:
            for r in rows_of[b][c * 6:(c + 1) * 6]:
                gidx[c, pos] = (_D + 1) * r
                pos += 1
        for s, b in enumerate(single[c * 9:(c + 1) * 9]):
            gidx[c, pos] = (_D + 1) * rows_of[b][0]
            sbins[c, s] = b
            pos += 1
        assert pos == _RPC
    return gidx.reshape(-1), sbins.reshape(-1), big[0], mid, single


_GIDX_NP, _SBINS_NP, _BIGBIN, _MIDBINS, _SINGLEBINS = _build_tables()


def _sc_body(diag_hbm, gidx_hbm, sbins_hbm, out_hbm,
             idx_v, sbins_v, vals_v, stage_v, sem):
    wid = lax.axis_index("s") * _NC + lax.axis_index("c")
    rc = wid // _NBC
    bc = wid - rc * _NBC
    boff = pl.multiple_of(bc * 16, 16)

    pltpu.sync_copy(gidx_hbm.at[pl.ds(pl.multiple_of(rc * _RPAD, 8), _RPAD)],
                    idx_v)
    pltpu.sync_copy(sbins_hbm.at[pl.ds(pl.multiple_of(rc * _SB, 8), _SB)],
                    sbins_v.at[pl.ds(0, _SB)])
    cp0 = pltpu.async_copy(diag_hbm.at[idx_v.at[pl.ds(0, 72)]],
                           vals_v.at[pl.ds(0, 72)], sem)
    cp1 = pltpu.async_copy(diag_hbm.at[idx_v.at[pl.ds(72, 72)]],
                           vals_v.at[pl.ds(72, 72)], sem)

    zeros = jnp.zeros((16,), jnp.float32)
    for b in _SINGLEBINS:
        stage_v[pl.ds(b * 16, 16)] = zeros

    cp0.wait()
    cp1.wait()

    def _lane(p):
        return jnp.abs(vals_v[p, pl.ds(boff, 16)])

    t0, t1, t2 = _lane(0), _lane(1), _lane(2)
    for p in range(3, 75):
        if p % 3 == 0:
            t0 = t0 + _lane(p)
        elif p % 3 == 1:
            t1 = t1 + _lane(p)
        else:
            t2 = t2 + _lane(p)
    stage_v[pl.ds(_BIGBIN * 16, 16)] = t0 + t1 + t2

    for q, b in enumerate(_MIDBINS):
        base = 75 + 6 * q
        tot = _lane(base)
        for p in range(base + 1, base + 6):
            tot = tot + _lane(p)
        stage_v[pl.ds(b * 16, 16)] = tot

    for s in range(21):
        b = sbins_v[pl.ds(s, 16)][0]
        stage_v[pl.ds(pl.multiple_of(b * 16, 16), 16)] = _lane(123 + s)

    pltpu.sync_copy(stage_v.at[pl.ds(0, _ACC)],
                    out_hbm.at[pl.ds(wid * _ACC, _ACC)])


@jax.jit
def _partial_measure(diag_tab, gidx, sbins):
    mesh = plsc.VectorSubcoreMesh(core_axis_name="c", subcore_axis_name="s",
                                  num_cores=_NC, num_subcores=_NS)
    parts = pl.kernel(
        _sc_body,
        out_type=jax.ShapeDtypeStruct((_NW * _ACC,), jnp.float32),
        mesh=mesh,
        compiler_params=pltpu.CompilerParams(needs_layout_passes=False,
                                             use_tc_tiling_on_sc=True),
        scratch_types=[
            pltpu.VMEM((_RPAD,), jnp.int32),
            pltpu.VMEM((_SB + 16,), jnp.int32),
            pltpu.VMEM((_RPAD, _B), jnp.float32),
            pltpu.VMEM(((_L + 1) * 16,), jnp.float32),
            pltpu.SemaphoreType.DMA,
        ],
    )(diag_tab, gidx, sbins)
    p = parts.reshape(_NRC, _NBC, _L, 16).sum(0)
    return p.transpose(0, 2, 1).reshape(_B, _L)


def kernel(rho):
    diag_tab = jnp.transpose(rho, (1, 2, 0)).reshape(_D * _D, _B)
    return _partial_measure(diag_tab, jnp.asarray(_GIDX_NP),
                            jnp.asarray(_SBINS_NP))


# speedup vs baseline: 1.0542x; 1.0176x over previous
import jax
import jax.numpy as jnp
import numpy as np
from jax import lax
from jax.experimental import pallas as pl
from jax.experimental.pallas import tpu as pltpu
from jax.experimental.pallas import tpu_sc as plsc

_M = 32
_N = 2
_SUBSET = 8
_B = 128
_D = 528
_L = 45

_NC, _NS = 2, 16
_NW = _NC * _NS
_NRC = 4
_NBC = 8
_RPC = _D // _NRC
_RPAD = 144
_SB = 24
_ACC = _L * 16


def _fock_states(m, n):
    if n == 0:
        yield (0,) * m
        return
    if m == 1:
        yield (n,)
        return
    for i in range(n + 1):
        for s in _fock_states(m - 1, n - i):
            yield (i,) + s


def _build_tables():
    all_states = list(_fock_states(_M, _N))
    reduced = []
    for i in range(max(0, _SUBSET - _M + _N), _N + 1):
        reduced += list(_fock_states(_SUBSET, i))
    binmap = np.array([reduced.index(s[:_SUBSET]) for s in all_states],
                      dtype=np.int64)
    rows_of = {b: list(np.where(binmap == b)[0]) for b in range(_L)}
    big = [b for b in range(_L) if len(rows_of[b]) == 300]
    mid = sorted(b for b in range(_L) if len(rows_of[b]) == 24)
    single = sorted(b for b in range(_L) if len(rows_of[b]) == 1)
    assert len(big) == 1 and len(mid) == 8 and len(single) == 36

    gidx = np.zeros((_NRC, _RPAD), np.int32)
    sbins = []
    for c in range(_NRC):
        pos = 0
        for r in rows_of[big[0]][c * 75:(c + 1) * 75]:
            gidx[c, pos] = (_D + 1) * r
            pos += 1
        for b in mid:
            for r in rows_of[b][c * 6:(c + 1) * 6]:
                gidx[c, pos] = (_D + 1) * r
                pos += 1
        sbins.append(single[c * 9:(c + 1) * 9])
        for b in sbins[-1]:
            gidx[c, pos] = (_D + 1) * rows_of[b][0]
            pos += 1
        assert pos == _RPC
    return gidx.reshape(-1), sbins, big[0], mid, single


_GIDX_NP, _SBINS, _BIGBIN, _MIDBINS, _SINGLEBINS = _build_tables()


def _sc_body(diag_hbm, gidx_hbm, out_hbm, idx_v, vals_v, stage_v, sem):
    wid = lax.axis_index("s") * _NC + lax.axis_index("c")
    rc = wid // _NBC
    bc = wid - rc * _NBC
    boff = pl.multiple_of(bc * 16, 16)

    pltpu.sync_copy(gidx_hbm.at[pl.ds(pl.multiple_of(rc * _RPAD, 8), _RPAD)],
                    idx_v)
    cp0 = pltpu.async_copy(diag_hbm.at[idx_v.at[pl.ds(0, 72)]],
                           vals_v.at[pl.ds(0, 72)], sem)
    cp1 = pltpu.async_copy(diag_hbm.at[idx_v.at[pl.ds(72, 72)]],
                           vals_v.at[pl.ds(72, 72)], sem)

    zeros = jnp.zeros((16,), jnp.float32)
    for b in _SINGLEBINS:
        stage_v[pl.ds(b * 16, 16)] = zeros

    cp0.wait()
    cp1.wait()

    def _lane(p):
        return jnp.abs(vals_v[p, pl.ds(boff, 16)])

    t0, t1, t2 = _lane(0), _lane(1), _lane(2)
    for p in range(3, 75):
        if p % 3 == 0:
            t0 = t0 + _lane(p)
        elif p % 3 == 1:
            t1 = t1 + _lane(p)
        else:
            t2 = t2 + _lane(p)
    stage_v[pl.ds(_BIGBIN * 16, 16)] = t0 + t1 + t2

    for q, b in enumerate(_MIDBINS):
        base = 75 + 6 * q
        tot = _lane(base)
        for p in range(base + 1, base + 6):
            tot = tot + _lane(p)
        stage_v[pl.ds(b * 16, 16)] = tot

    for c in range(_NRC):
        @pl.when(rc == c)
        def _(c=c):
            for s, b in enumerate(_SBINS[c]):
                stage_v[pl.ds(b * 16, 16)] = _lane(123 + s)

    pltpu.sync_copy(stage_v.at[pl.ds(0, _ACC)],
                    out_hbm.at[pl.ds(wid * _ACC, _ACC)])


@jax.jit
def _partial_measure(diag_tab, gidx):
    mesh = plsc.VectorSubcoreMesh(core_axis_name="c", subcore_axis_name="s",
                                  num_cores=_NC, num_subcores=_NS)
    parts = pl.kernel(
        _sc_body,
        out_type=jax.ShapeDtypeStruct((_NW * _ACC,), jnp.float32),
        mesh=mesh,
        compiler_params=pltpu.CompilerParams(needs_layout_passes=False,
                                             use_tc_tiling_on_sc=True),
        scratch_types=[
            pltpu.VMEM((_RPAD,), jnp.int32),
            pltpu.VMEM((_RPAD, _B), jnp.float32),
            pltpu.VMEM((_ACC,), jnp.float32),
            pltpu.SemaphoreType.DMA,
        ],
    )(diag_tab, gidx)
    p = parts.reshape(_NRC, _NBC, _L, 16).sum(0)
    return p.transpose(0, 2, 1).reshape(_B, _L)


def kernel(rho):
    diag_tab = jnp.transpose(rho, (1, 2, 0)).reshape(_D * _D, _B)
    return _partial_measure(diag_tab, jnp.asarray(_GIDX_NP))


# speedup vs baseline: 1.5512x; 1.4715x over previous
import jax
import jax.numpy as jnp
import numpy as np
from jax import lax
from jax.experimental import pallas as pl
from jax.experimental.pallas import tpu as pltpu
from jax.experimental.pallas import tpu_sc as plsc

_M = 32
_N = 2
_SUBSET = 8
_B = 128
_D = 528
_L = 45

_NC, _NS = 2, 16
_NW = _NC * _NS
_NRC = 4
_NBC = 8
_RPC = _D // _NRC
_ACC = _L * 16


def _fock_states(m, n):
    if n == 0:
        yield (0,) * m
        return
    if m == 1:
        yield (n,)
        return
    for i in range(n + 1):
        for s in _fock_states(m - 1, n - i):
            yield (i,) + s


def _build_binmap():
    all_states = list(_fock_states(_M, _N))
    reduced = []
    for i in range(max(0, _SUBSET - _M + _N), _N + 1):
        reduced += list(_fock_states(_SUBSET, i))
    return [reduced.index(s[:_SUBSET]) for s in all_states]


_BINMAP = _build_binmap()

_CHUNK_BASES = [16 * k for k in range(8)] + [116]

_GROUPS = []
for _c in range(_NRC):
    g = {}
    for _r in range(_RPC):
        _p = _r if _r < 128 else _r + 12
        g.setdefault(_BINMAP[_c * _RPC + _r], []).append(_p)
    _GROUPS.append(g)


def _sc_body(diag_hbm, out_hbm, vals_v, stage_v, sem):
    wid = lax.axis_index("s") * _NC + lax.axis_index("c")
    rc = wid // _NBC
    bc = wid - rc * _NBC
    boff = pl.multiple_of(bc * 16, 16)

    lane = lax.iota(jnp.int32, 16)
    base = rc * ((_D + 1) * _RPC)
    copies = []
    for k, cb in enumerate(_CHUNK_BASES):
        idx = base + ((_D + 1) * cb + (_D + 1) * lane)
        copies.append(
            pltpu.async_copy(diag_hbm.at[idx],
                             vals_v.at[pl.ds(16 * k, 16)], sem))
    for cp in copies:
        cp.wait()

    zeros = jnp.zeros((16,), jnp.float32)
    for c in range(_NRC):
        @pl.when(rc == c)
        def _(c=c):
            for b in range(_L):
                rows = _GROUPS[c].get(b)
                if rows is None:
                    stage_v[pl.ds(b * 16, 16)] = zeros
                    continue
                accs = []
                for j, p in enumerate(rows):
                    v = jnp.abs(vals_v[p, pl.ds(boff, 16)])
                    if j < 3:
                        accs.append(v)
                    else:
                        accs[j % 3] = accs[j % 3] + v
                tot = accs[0]
                for a in accs[1:]:
                    tot = tot + a
                stage_v[pl.ds(b * 16, 16)] = tot

    pltpu.sync_copy(stage_v, out_hbm.at[pl.ds(wid * _ACC, _ACC)])


@jax.jit
def _partial_measure(diag_tab):
    mesh = plsc.VectorSubcoreMesh(core_axis_name="c", subcore_axis_name="s",
                                  num_cores=_NC, num_subcores=_NS)
    parts = pl.kernel(
        _sc_body,
        out_type=jax.ShapeDtypeStruct((_NW * _ACC,), jnp.float32),
        mesh=mesh,
        compiler_params=pltpu.CompilerParams(needs_layout_passes=False,
                                             use_tc_tiling_on_sc=True),
        scratch_types=[
            pltpu.VMEM((144, _B), jnp.float32),
            pltpu.VMEM((_ACC,), jnp.float32),
            pltpu.SemaphoreType.DMA,
        ],
    )(diag_tab)
    p = parts.reshape(_NRC, _NBC, _L, 16).sum(0)
    return p.transpose(0, 2, 1).reshape(_B, _L)


def kernel(rho):
    diag_tab = jnp.transpose(rho, (1, 2, 0)).reshape(_D * _D, _B)
    return _partial_measure(diag_tab)
